# baseline (speedup 1.0000x reference)
import functools

import jax
import jax.numpy as jnp
from jax import lax
from jax.experimental import pallas as pl
from jax.experimental.pallas import tpu as pltpu
from jax.experimental.pallas import tpu_sc as plsc

DIM = 256
K = 8192
COMMIT_WEIGHT = 0.25


TB = 128


def _argmin_body(x_ref, cb_ref, ind_ref, c2_ref):
    i = pl.program_id(0)

    @pl.when(i == 0)
    def _():
        cb = cb_ref[...]
        c2_ref[...] = jnp.sum(cb * cb, axis=1, keepdims=True)

    x = x_ref[...]
    xsq = x * x
    ones = jnp.ones((8, DIM), jnp.float32)
    f2 = lax.dot_general(
        ones, xsq, (((1,), (1,)), ((), ())),
        preferred_element_type=jnp.float32)[0:1, :]
    dots = lax.dot_general(
        cb_ref[...], x, (((1,), (1,)), ((), ())),
        preferred_element_type=jnp.float32)
    t = (f2 - 2.0 * dots) + c2_ref[...]
    mn = jnp.min(t, axis=0, keepdims=True)
    ids = lax.broadcasted_iota(jnp.int32, (K, TB), 0)
    ind = jnp.min(jnp.where(t == mn, ids, K), axis=0)
    ind_ref[...] = ind


def _argmin_call(flat, codebook):
    bn = flat.shape[0]
    return pl.pallas_call(
        _argmin_body,
        grid=(bn // TB,),
        in_specs=[
            pl.BlockSpec((TB, DIM), lambda i: (i, 0)),
            pl.BlockSpec((K, DIM), lambda i: (0, 0)),
        ],
        out_specs=pl.BlockSpec((TB,), lambda i: (i,)),
        out_shape=jax.ShapeDtypeStruct((bn,), jnp.int32),
        scratch_shapes=[pltpu.VMEM((K, 1), jnp.float32)],
    )(flat, codebook)


NC, NS = 2, 16
NW = NC * NS
CH = 128


def _gather_call(codebook, ind, bn):
    chunks = bn // (NW * CH)
    mesh = plsc.VectorSubcoreMesh(core_axis_name="c", subcore_axis_name="s",
                                  num_cores=NC, num_subcores=NS)

    @functools.partial(
        pl.kernel,
        out_type=jax.ShapeDtypeStruct((bn, DIM), jnp.float32),
        mesh=mesh,
        scratch_types=[
            pltpu.VMEM((CH,), jnp.int32),
            pltpu.VMEM((CH, DIM), jnp.float32),
            pltpu.SemaphoreType.DMA,
        ],
    )
    def gather(cb_hbm, idx_hbm, out_hbm, idx_v, rows_v, sem):
        wid = lax.axis_index("s") * NC + lax.axis_index("c")
        for j in range(chunks):
            base = (wid * chunks + j) * CH
            pltpu.sync_copy(idx_hbm.at[pl.ds(base, CH)], idx_v)
            pltpu.async_copy(cb_hbm.at[idx_v], rows_v, sem).wait()
            pltpu.sync_copy(rows_v, out_hbm.at[pl.ds(base, CH)])

    return gather(codebook, ind)


TB3 = 512


def _st_loss_body(x_ref, q_ref, qst_ref, loss_ref):
    i = pl.program_id(0)
    x = x_ref[...]
    d = q_ref[...] - x
    qst_ref[...] = x + d
    part = jnp.sum(d * d)

    @pl.when(i == 0)
    def _():
        loss_ref[0, 0] = 0.0

    loss_ref[0, 0] += part

    @pl.when(i == pl.num_programs(0) - 1)
    def _():
        n = pl.num_programs(0) * TB3 * DIM
        loss_ref[0, 0] = COMMIT_WEIGHT * (loss_ref[0, 0] / n)


def _st_loss_call(flat, quant):
    bn = flat.shape[0]
    return pl.pallas_call(
        _st_loss_body,
        grid=(bn // TB3,),
        in_specs=[
            pl.BlockSpec((TB3, DIM), lambda i: (i, 0)),
            pl.BlockSpec((TB3, DIM), lambda i: (i, 0)),
        ],
        out_specs=[
            pl.BlockSpec((TB3, DIM), lambda i: (i, 0)),
            pl.BlockSpec(memory_space=pltpu.SMEM),
        ],
        out_shape=[
            jax.ShapeDtypeStruct((bn, DIM), jnp.float32),
            jax.ShapeDtypeStruct((1, 1), jnp.float32),
        ],
    )(flat, quant)


def kernel(x, codebook):
    B, N, D = x.shape
    flat = x.reshape(-1, D)
    ind = _argmin_call(flat, codebook)
    quant = _gather_call(codebook, ind, flat.shape[0])
    qst, loss = _st_loss_call(flat, quant)
    return (qst.reshape(B, N, D), ind.reshape(B, N),
            loss[0, 0])
